# Initial kernel scaffold; baseline (speedup 1.0000x reference)
import functools

import jax
import jax.numpy as jnp
from jax import lax
from jax.experimental import pallas as pl
from jax.experimental.pallas import tpu as pltpu
from jax.experimental.pallas import tpu_sc as plsc

N = 10000
E = 160000
D = 256
DH = 128
NT = 16
NC = 2
E_PAD = 163840
EPT_DEG = E_PAD // (NC * NT)
EPT_AGG = E_PAD // NT
CHUNK = 128
NPT = N // NT

_mesh = plsc.VectorSubcoreMesh(core_axis_name="c", subcore_axis_name="s")


def _deg_body(col_hbm, ew_hbm, out_hbm, deg_v, col_v, ew_v):
    c = lax.axis_index("c")
    s = lax.axis_index("s")
    wid = s * NC + c
    base = wid * EPT_DEG
    pltpu.sync_copy(col_hbm.at[pl.ds(base, EPT_DEG)], col_v)
    pltpu.sync_copy(ew_hbm.at[pl.ds(base, EPT_DEG)], ew_v)

    zeros = jnp.zeros((16,), jnp.float32)

    def zbody(i, carry):
        deg_v[pl.ds(i * 16, 16)] = zeros
        return carry

    lax.fori_loop(0, N // 16, zbody, 0)

    def ebody(i, carry):
        idx = col_v[pl.ds(i * 16, 16)]
        w = ew_v[pl.ds(i * 16, 16)]
        plsc.addupdate_scatter(deg_v, [idx], w)
        return carry

    lax.fori_loop(0, EPT_DEG // 16, ebody, 0)
    pltpu.sync_copy(deg_v, out_hbm.at[wid])


_deg_call = functools.partial(
    pl.kernel,
    out_type=jax.ShapeDtypeStruct((NC * NT, N), jnp.float32),
    mesh=_mesh,
    scratch_types=[
        pltpu.VMEM((N,), jnp.float32),
        pltpu.VMEM((EPT_DEG,), jnp.int32),
        pltpu.VMEM((EPT_DEG,), jnp.float32),
    ],
)(_deg_body)


def _mm_body(degpt_ref, x_ref, w_ref, dis_ref, hcat_ref):
    deg = jnp.sum(degpt_ref[...], axis=1, keepdims=True) + 1.0
    dis = lax.rsqrt(deg)
    dis_ref[...] = dis
    h = jnp.dot(x_ref[...], w_ref[...], preferred_element_type=jnp.float32)
    hp = h * dis
    hcat_ref[0] = hp[:, 0:DH]
    hcat_ref[1] = hp[:, DH:D]


_mm_call = pl.pallas_call(
    _mm_body,
    out_shape=[
        jax.ShapeDtypeStruct((N, 1), jnp.float32),
        jax.ShapeDtypeStruct((2, N, DH), jnp.float32),
    ],
    compiler_params=pltpu.CompilerParams(vmem_limit_bytes=100 * 1024 * 1024),
)


def _agg_body(hcat_hbm, row_hbm, col_hbm, ew_hbm, agg_hbm,
              idx_v, cidx_v, row_v, col_v, ew_v, gbuf, shared_agg, sem):
    c = lax.axis_index("c")
    s = lax.axis_index("s")
    node_off = c * N
    base = s * EPT_AGG

    pltpu.sync_copy(row_hbm.at[pl.ds(base, EPT_AGG)], row_v)
    pltpu.sync_copy(col_hbm.at[pl.ds(base, EPT_AGG)], col_v)
    pltpu.sync_copy(ew_hbm.at[pl.ds(base, EPT_AGG)], ew_v)

    pltpu.sync_copy(hcat_hbm.at[pl.ds(node_off + s * NPT, NPT)],
                    shared_agg.at[pl.ds(s * NPT, NPT)])
    plsc.subcore_barrier()

    def chunk_body(k, carry):
        off = k * CHUNK

        def ibody(i, icarry):
            r = row_v[pl.ds(off + i * 16, 16)]
            idx_v[pl.ds(i * 16, 16)] = r + node_off
            cidx_v[pl.ds(i * 16, 16)] = col_v[pl.ds(off + i * 16, 16)]
            return icarry

        lax.fori_loop(0, CHUNK // 16, ibody, 0)

        pltpu.async_copy(hcat_hbm.at[idx_v], gbuf, sem).wait()

        def ebody(j, ecarry):
            w = ew_v[off + j]
            for f in range(DH // 16):
                sl = pl.ds(f * 16, 16)
                gbuf[j, sl] = gbuf[j, sl] * w
            return ecarry

        lax.fori_loop(0, CHUNK, ebody, 0)

        pltpu.sync_copy(gbuf, shared_agg.at[cidx_v], add=True)
        return carry

    lax.fori_loop(0, EPT_AGG // CHUNK, chunk_body, 0)
    plsc.subcore_barrier()

    pltpu.sync_copy(shared_agg.at[pl.ds(s * NPT, NPT)],
                    agg_hbm.at[pl.ds(node_off + s * NPT, NPT)])


_agg_call = functools.partial(
    pl.kernel,
    out_type=jax.ShapeDtypeStruct((NC * N, DH), jnp.float32),
    mesh=_mesh,
    scratch_types=[
        pltpu.VMEM((CHUNK,), jnp.int32),
        pltpu.VMEM((CHUNK,), jnp.int32),
        pltpu.VMEM((EPT_AGG,), jnp.int32),
        pltpu.VMEM((EPT_AGG,), jnp.int32),
        pltpu.VMEM((EPT_AGG,), jnp.float32),
        pltpu.VMEM((CHUNK, DH), jnp.float32),
        pltpu.VMEM_SHARED((N, DH), jnp.float32),
        pltpu.SemaphoreType.DMA,
    ],
)(_agg_body)


def _bn_body(agg_ref, dis_ref, x_ref, b_ref, gamma_ref, beta_ref, out_ref):
    agg = jnp.concatenate([agg_ref[0:N, :], agg_ref[N:2 * N, :]], axis=1)
    y = agg * dis_ref[...] + b_ref[...]
    mean = jnp.mean(y, axis=0, keepdims=True)
    yc = y - mean
    var = jnp.mean(yc * yc, axis=0, keepdims=True)
    out_ref[...] = yc * lax.rsqrt(var + 1e-5) * gamma_ref[...] \
        + beta_ref[...] + x_ref[...]


_bn_call = pl.pallas_call(
    _bn_body,
    out_shape=jax.ShapeDtypeStruct((N, D), jnp.float32),
    compiler_params=pltpu.CompilerParams(vmem_limit_bytes=100 * 1024 * 1024),
)


def kernel(x, edge_index, edge_weight, W, b, gamma, beta):
    row = edge_index[0].astype(jnp.int32)
    col = edge_index[1].astype(jnp.int32)
    ew = edge_weight.astype(jnp.float32)
    pad = E_PAD - E
    row_p = jnp.concatenate([row, jnp.zeros((pad,), jnp.int32)])
    col_p = jnp.concatenate([col, jnp.zeros((pad,), jnp.int32)])
    ew_p = jnp.concatenate([ew, jnp.zeros((pad,), jnp.float32)])

    degp = _deg_call(col_p, ew_p)
    degpt = degp.T
    dis, hcat = _mm_call(degpt, x, W)
    hflat = hcat.reshape(NC * N, DH)
    agg = _agg_call(hflat, row_p, col_p, ew_p)
    out = _bn_call(agg, dis, x,
                   b.reshape(1, D), gamma.reshape(1, D), beta.reshape(1, D))
    return out


# speedup vs baseline: 8.7335x; 8.7335x over previous
import functools

import jax
import jax.numpy as jnp
from jax import lax
from jax.experimental import pallas as pl
from jax.experimental.pallas import tpu as pltpu
from jax.experimental.pallas import tpu_sc as plsc

N = 10000
E = 160000
D = 256
DH = 128
NT = 16
NC = 2
E_PAD = 163840
EPT_DEG = E_PAD // (NC * NT)
EPT_AGG = E_PAD // NT
CHUNK = 128
NPT = 624
NREM = N - NT * NPT

_mesh = plsc.VectorSubcoreMesh(core_axis_name="c", subcore_axis_name="s")


def _deg_body(col_hbm, ew_hbm, out_hbm, deg_v, col_v, ew_v):
    c = lax.axis_index("c")
    s = lax.axis_index("s")
    wid = s * NC + c
    base = wid * EPT_DEG
    pltpu.sync_copy(col_hbm.at[pl.ds(base, EPT_DEG)], col_v)
    pltpu.sync_copy(ew_hbm.at[pl.ds(base, EPT_DEG)], ew_v)

    zeros = jnp.zeros((16,), jnp.float32)

    def zbody(i, carry):
        deg_v[pl.ds(i * 16, 16)] = zeros
        return carry

    lax.fori_loop(0, N // 16, zbody, 0)

    def ebody(i, carry):
        idx = col_v[pl.ds(i * 16, 16)]
        w = ew_v[pl.ds(i * 16, 16)]
        plsc.addupdate_scatter(deg_v, [idx], w)
        return carry

    lax.fori_loop(0, EPT_DEG // 16, ebody, 0)
    pltpu.sync_copy(deg_v, out_hbm.at[wid])


_deg_call = functools.partial(
    pl.kernel,
    out_type=jax.ShapeDtypeStruct((NC * NT, N), jnp.float32),
    mesh=_mesh,
    compiler_params=pltpu.CompilerParams(needs_layout_passes=False),
    scratch_types=[
        pltpu.VMEM((N,), jnp.float32),
        pltpu.VMEM((EPT_DEG,), jnp.int32),
        pltpu.VMEM((EPT_DEG,), jnp.float32),
    ],
)(_deg_body)


def _mm_body(degpt_ref, x_ref, w_ref, dis_ref, hcat_ref):
    deg = jnp.sum(degpt_ref[...], axis=1, keepdims=True) + 1.0
    dis = lax.rsqrt(deg)
    dis_ref[...] = dis
    h = jnp.dot(x_ref[...], w_ref[...], preferred_element_type=jnp.float32)
    hp = h * dis
    hcat_ref[0] = hp[:, 0:DH]
    hcat_ref[1] = hp[:, DH:D]


_mm_call = pl.pallas_call(
    _mm_body,
    out_shape=[
        jax.ShapeDtypeStruct((N, 1), jnp.float32),
        jax.ShapeDtypeStruct((2, N, DH), jnp.float32),
    ],
    compiler_params=pltpu.CompilerParams(vmem_limit_bytes=100 * 1024 * 1024),
)


def _agg_body(hcat_hbm, row_hbm, col_hbm, ew_hbm, agg_hbm,
              idx_v, cidx_v, row_v, col_v, ew_v, gbuf, shared_agg, sem):
    c = lax.axis_index("c")
    s = lax.axis_index("s")
    node_off = c * N
    base = s * EPT_AGG

    pltpu.sync_copy(row_hbm.at[pl.ds(base, EPT_AGG)], row_v)
    pltpu.sync_copy(col_hbm.at[pl.ds(base, EPT_AGG)], col_v)
    pltpu.sync_copy(ew_hbm.at[pl.ds(base, EPT_AGG)], ew_v)

    pltpu.sync_copy(hcat_hbm.at[pl.ds(node_off + s * NPT, NPT)],
                    shared_agg.at[pl.ds(s * NPT, NPT)])

    @pl.when(s == 0)
    def _init_rem():
        pltpu.sync_copy(hcat_hbm.at[pl.ds(node_off + NT * NPT, NREM)],
                        shared_agg.at[pl.ds(NT * NPT, NREM)])

    plsc.subcore_barrier()

    def chunk_body(k, carry):
        off = k * CHUNK

        def ibody(i, icarry):
            r = row_v[pl.ds(off + i * 16, 16)]
            idx_v[pl.ds(i * 16, 16)] = r + node_off
            cidx_v[pl.ds(i * 16, 16)] = col_v[pl.ds(off + i * 16, 16)]
            return icarry

        lax.fori_loop(0, CHUNK // 16, ibody, 0)

        pltpu.async_copy(hcat_hbm.at[idx_v], gbuf, sem).wait()

        def ebody(g, ecarry):
            wv = ew_v[pl.ds(off + g * 16, 16)]
            jb = g * 16
            for l in range(16):
                w = wv[l]
                for f in range(DH // 16):
                    sl = pl.ds(f * 16, 16)
                    gbuf[jb + l, sl] = gbuf[jb + l, sl] * w
            return ecarry

        lax.fori_loop(0, CHUNK // 16, ebody, 0)

        pltpu.sync_copy(gbuf, shared_agg.at[cidx_v], add=True)
        return carry

    lax.fori_loop(0, EPT_AGG // CHUNK, chunk_body, 0)
    plsc.subcore_barrier()

    pltpu.sync_copy(shared_agg.at[pl.ds(s * NPT, NPT)],
                    agg_hbm.at[pl.ds(node_off + s * NPT, NPT)])

    @pl.when(s == 0)
    def _out_rem():
        pltpu.sync_copy(shared_agg.at[pl.ds(NT * NPT, NREM)],
                        agg_hbm.at[pl.ds(node_off + NT * NPT, NREM)])


_agg_call = functools.partial(
    pl.kernel,
    out_type=jax.ShapeDtypeStruct((NC * N, DH), jnp.float32),
    mesh=_mesh,
    scratch_types=[
        pltpu.VMEM((CHUNK,), jnp.int32),
        pltpu.VMEM((CHUNK,), jnp.int32),
        pltpu.VMEM((EPT_AGG,), jnp.int32),
        pltpu.VMEM((EPT_AGG,), jnp.int32),
        pltpu.VMEM((EPT_AGG,), jnp.float32),
        pltpu.VMEM((CHUNK, DH), jnp.float32),
        pltpu.VMEM_SHARED((N, DH), jnp.float32),
        pltpu.SemaphoreType.DMA,
    ],
)(_agg_body)


def _bn_body(agg_ref, dis_ref, x_ref, b_ref, gamma_ref, beta_ref, out_ref):
    agg = jnp.concatenate([agg_ref[0:N, :], agg_ref[N:2 * N, :]], axis=1)
    y = agg * dis_ref[...] + b_ref[...]
    mean = jnp.mean(y, axis=0, keepdims=True)
    yc = y - mean
    var = jnp.mean(yc * yc, axis=0, keepdims=True)
    out_ref[...] = yc * lax.rsqrt(var + 1e-5) * gamma_ref[...] \
        + beta_ref[...] + x_ref[...]


_bn_call = pl.pallas_call(
    _bn_body,
    out_shape=jax.ShapeDtypeStruct((N, D), jnp.float32),
    compiler_params=pltpu.CompilerParams(vmem_limit_bytes=100 * 1024 * 1024),
)


def kernel(x, edge_index, edge_weight, W, b, gamma, beta):
    row = edge_index[0].astype(jnp.int32)
    col = edge_index[1].astype(jnp.int32)
    ew = edge_weight.astype(jnp.float32)
    pad = E_PAD - E
    row_p = jnp.concatenate([row, jnp.zeros((pad,), jnp.int32)])
    col_p = jnp.concatenate([col, jnp.zeros((pad,), jnp.int32)])
    ew_p = jnp.concatenate([ew, jnp.zeros((pad,), jnp.float32)])

    degp = _deg_call(col_p, ew_p)
    degpt = degp.T
    dis, hcat = _mm_call(degpt, x, W)
    hflat = hcat.reshape(NC * N, DH)
    agg = _agg_call(hflat, row_p, col_p, ew_p)
    out = _bn_call(agg, dis, x,
                   b.reshape(1, D), gamma.reshape(1, D), beta.reshape(1, D))
    return out


# speedup vs baseline: 10.0399x; 1.1496x over previous
import functools

import jax
import jax.numpy as jnp
from jax import lax
from jax.experimental import pallas as pl
from jax.experimental.pallas import tpu as pltpu
from jax.experimental.pallas import tpu_sc as plsc

N = 10000
E = 160000
D = 256
DH = 128
NT = 16
NC = 2
E_PAD = 163840
EPT_DEG = E_PAD // (NC * NT)
EPT_AGG = E_PAD // NT
CHUNK = 80
NPT = 624
NREM = N - NT * NPT

_mesh = plsc.VectorSubcoreMesh(core_axis_name="c", subcore_axis_name="s")


def _deg_body(col_hbm, ew_hbm, out_hbm, deg_v, col_v, ew_v):
    c = lax.axis_index("c")
    s = lax.axis_index("s")
    wid = s * NC + c
    base = wid * EPT_DEG
    pltpu.sync_copy(col_hbm.at[pl.ds(base, EPT_DEG)], col_v)
    pltpu.sync_copy(ew_hbm.at[pl.ds(base, EPT_DEG)], ew_v)

    zeros = jnp.zeros((16,), jnp.float32)

    def zbody(i, carry):
        deg_v[pl.ds(i * 16, 16)] = zeros
        return carry

    lax.fori_loop(0, N // 16, zbody, 0)

    def ebody(i, carry):
        idx = col_v[pl.ds(i * 16, 16)]
        w = ew_v[pl.ds(i * 16, 16)]
        plsc.addupdate_scatter(deg_v, [idx], w)
        return carry

    lax.fori_loop(0, EPT_DEG // 16, ebody, 0)
    pltpu.sync_copy(deg_v, out_hbm.at[wid])


_deg_call = functools.partial(
    pl.kernel,
    out_type=jax.ShapeDtypeStruct((NC * NT, N), jnp.float32),
    mesh=_mesh,
    compiler_params=pltpu.CompilerParams(needs_layout_passes=False),
    scratch_types=[
        pltpu.VMEM((N,), jnp.float32),
        pltpu.VMEM((EPT_DEG,), jnp.int32),
        pltpu.VMEM((EPT_DEG,), jnp.float32),
    ],
)(_deg_body)


def _mm_body(degpt_ref, x_ref, w_ref, dis_ref, hcat_ref):
    deg = jnp.sum(degpt_ref[...], axis=1, keepdims=True) + 1.0
    dis = lax.rsqrt(deg)
    dis_ref[...] = dis
    h = jnp.dot(x_ref[...], w_ref[...], preferred_element_type=jnp.float32)
    hp = h * dis
    hcat_ref[0] = hp[:, 0:DH]
    hcat_ref[1] = hp[:, DH:D]


_mm_call = pl.pallas_call(
    _mm_body,
    out_shape=[
        jax.ShapeDtypeStruct((N, 1), jnp.float32),
        jax.ShapeDtypeStruct((2, N, DH), jnp.float32),
    ],
    compiler_params=pltpu.CompilerParams(vmem_limit_bytes=100 * 1024 * 1024),
)


NBUF = 4
NGRP = EPT_AGG // (NBUF * CHUNK)


def _agg_body(hcat_hbm, row_hbm, col_hbm, ew_hbm, agg_hbm,
              idx_v, cidx_v, ew_v, gbuf, shared_agg,
              isem, csem, wsem, gsem, ssem):
    c = lax.axis_index("c")
    s = lax.axis_index("s")
    base = s * EPT_AGG
    htab = hcat_hbm.at[c]
    atab = agg_hbm.at[c]

    pltpu.sync_copy(htab.at[pl.ds(s * NPT, NPT)],
                    shared_agg.at[pl.ds(s * NPT, NPT)])

    @pl.when(s == 0)
    def _init_rem():
        pltpu.sync_copy(htab.at[pl.ds(NT * NPT, NREM)],
                        shared_agg.at[pl.ds(NT * NPT, NREM)])

    plsc.subcore_barrier()

    def group_body(g, carry):
        goff = base + g * (NBUF * CHUNK)

        for b in range(NBUF):
            off = goff + b * CHUNK

            @pl.when(g > 0)
            def _drain_prev(b=b):
                pltpu.make_async_copy(
                    gbuf.at[b], shared_agg.at[cidx_v.at[b]], ssem.at[b]
                ).wait()

            pltpu.async_copy(row_hbm.at[pl.ds(off, CHUNK)], idx_v.at[b],
                             isem.at[b])
            pltpu.async_copy(col_hbm.at[pl.ds(off, CHUNK)], cidx_v.at[b],
                             csem.at[b])
            pltpu.async_copy(ew_hbm.at[pl.ds(off, CHUNK)], ew_v.at[b],
                             wsem.at[b])

        for b in range(NBUF):
            pltpu.make_async_copy(row_hbm.at[pl.ds(base, CHUNK)],
                                  idx_v.at[b], isem.at[b]).wait()
            pltpu.async_copy(htab.at[idx_v.at[b]], gbuf.at[b], gsem.at[b])

        for b in range(NBUF):
            pltpu.make_async_copy(htab.at[idx_v.at[b]], gbuf.at[b],
                                  gsem.at[b]).wait()
            pltpu.make_async_copy(ew_hbm.at[pl.ds(base, CHUNK)],
                                  ew_v.at[b], wsem.at[b]).wait()

            def ebody(q, ecarry, b=b):
                wv = ew_v[b, pl.ds(q * 16, 16)]
                jb = q * 16
                for l in range(16):
                    w = wv[l]
                    for f in range(DH // 16):
                        sl = pl.ds(f * 16, 16)
                        gbuf[b, jb + l, sl] = gbuf[b, jb + l, sl] * w
                return ecarry

            lax.fori_loop(0, CHUNK // 16, ebody, 0)
            pltpu.make_async_copy(col_hbm.at[pl.ds(base, CHUNK)],
                                  cidx_v.at[b], csem.at[b]).wait()
            pltpu.async_copy(gbuf.at[b], shared_agg.at[cidx_v.at[b]],
                             ssem.at[b], add=True)
        return carry

    lax.fori_loop(0, NGRP, group_body, 0)

    for b in range(NBUF):
        pltpu.make_async_copy(
            gbuf.at[b], shared_agg.at[cidx_v.at[b]], ssem.at[b]).wait()
    plsc.subcore_barrier()

    pltpu.sync_copy(shared_agg.at[pl.ds(s * NPT, NPT)],
                    atab.at[pl.ds(s * NPT, NPT)])

    @pl.when(s == 0)
    def _out_rem():
        pltpu.sync_copy(shared_agg.at[pl.ds(NT * NPT, NREM)],
                        atab.at[pl.ds(NT * NPT, NREM)])


_agg_call = functools.partial(
    pl.kernel,
    out_type=jax.ShapeDtypeStruct((NC, N, DH), jnp.float32),
    mesh=_mesh,
    compiler_params=pltpu.CompilerParams(needs_layout_passes=False),
    scratch_types=[
        pltpu.VMEM((NBUF, CHUNK), jnp.int32),
        pltpu.VMEM((NBUF, CHUNK), jnp.int32),
        pltpu.VMEM((NBUF, CHUNK), jnp.float32),
        pltpu.VMEM((NBUF, CHUNK, DH), jnp.float32),
        pltpu.VMEM_SHARED((N, DH), jnp.float32),
        pltpu.SemaphoreType.DMA((NBUF,)),
        pltpu.SemaphoreType.DMA((NBUF,)),
        pltpu.SemaphoreType.DMA((NBUF,)),
        pltpu.SemaphoreType.DMA((NBUF,)),
        pltpu.SemaphoreType.DMA((NBUF,)),
    ],
)(_agg_body)


def _bn_body(agg_ref, dis_ref, x_ref, b_ref, gamma_ref, beta_ref, out_ref):
    agg = jnp.concatenate([agg_ref[0:N, :], agg_ref[N:2 * N, :]], axis=1)
    y = agg * dis_ref[...] + b_ref[...]
    mean = jnp.mean(y, axis=0, keepdims=True)
    yc = y - mean
    var = jnp.mean(yc * yc, axis=0, keepdims=True)
    out_ref[...] = yc * lax.rsqrt(var + 1e-5) * gamma_ref[...] \
        + beta_ref[...] + x_ref[...]


_bn_call = pl.pallas_call(
    _bn_body,
    out_shape=jax.ShapeDtypeStruct((N, D), jnp.float32),
    compiler_params=pltpu.CompilerParams(vmem_limit_bytes=100 * 1024 * 1024),
)


def kernel(x, edge_index, edge_weight, W, b, gamma, beta):
    row = edge_index[0].astype(jnp.int32)
    col = edge_index[1].astype(jnp.int32)
    ew = edge_weight.astype(jnp.float32)
    pad = E_PAD - E
    row_p = jnp.concatenate([row, jnp.zeros((pad,), jnp.int32)])
    col_p = jnp.concatenate([col, jnp.zeros((pad,), jnp.int32)])
    ew_p = jnp.concatenate([ew, jnp.zeros((pad,), jnp.float32)])

    degp = _deg_call(col_p, ew_p)
    degpt = degp.T
    dis, hcat = _mm_call(degpt, x, W)
    agg = _agg_call(hcat, row_p, col_p, ew_p)
    agg = agg.reshape(NC * N, DH)
    out = _bn_call(agg, dis, x,
                   b.reshape(1, D), gamma.reshape(1, D), beta.reshape(1, D))
    return out


# speedup vs baseline: 10.7147x; 1.0672x over previous
import functools

import jax
import jax.numpy as jnp
from jax import lax
from jax.experimental import pallas as pl
from jax.experimental.pallas import tpu as pltpu
from jax.experimental.pallas import tpu_sc as plsc

N = 10000
E = 160000
D = 256
DH = 128
NT = 16
NC = 2
E_PAD = 163840
EPT_DEG = E_PAD // (NC * NT)
EPT_AGG = E_PAD // NT
CHUNK = 80
NPT = 624
NREM = N - NT * NPT

_mesh = plsc.VectorSubcoreMesh(core_axis_name="c", subcore_axis_name="s")


def _deg_body(col_hbm, ew_hbm, out_hbm, deg_v, col_v, ew_v):
    c = lax.axis_index("c")
    s = lax.axis_index("s")
    wid = s * NC + c
    base = wid * EPT_DEG
    pltpu.sync_copy(col_hbm.at[pl.ds(base, EPT_DEG)], col_v)
    pltpu.sync_copy(ew_hbm.at[pl.ds(base, EPT_DEG)], ew_v)

    zeros = jnp.zeros((16,), jnp.float32)

    def zbody(i, carry):
        deg_v[pl.ds(i * 16, 16)] = zeros
        return carry

    lax.fori_loop(0, N // 16, zbody, 0)

    def ebody(i, carry):
        idx = col_v[pl.ds(i * 16, 16)]
        w = ew_v[pl.ds(i * 16, 16)]
        plsc.addupdate_scatter(deg_v, [idx], w)
        return carry

    lax.fori_loop(0, EPT_DEG // 16, ebody, 0)
    pltpu.sync_copy(deg_v, out_hbm.at[wid])


_deg_call = functools.partial(
    pl.kernel,
    out_type=jax.ShapeDtypeStruct((NC * NT, N), jnp.float32),
    mesh=_mesh,
    compiler_params=pltpu.CompilerParams(needs_layout_passes=False),
    scratch_types=[
        pltpu.VMEM((N,), jnp.float32),
        pltpu.VMEM((EPT_DEG,), jnp.int32),
        pltpu.VMEM((EPT_DEG,), jnp.float32),
    ],
)(_deg_body)


def _mm_body(degpt_ref, x_ref, w_ref, dis_ref, hcat_ref):
    deg = jnp.sum(degpt_ref[...], axis=1, keepdims=True) + 1.0
    dis = lax.rsqrt(deg)
    dis_ref[...] = dis
    h = jnp.dot(x_ref[...], w_ref[...], preferred_element_type=jnp.float32)
    hp = h * dis
    hcat_ref[0] = hp[:, 0:DH]
    hcat_ref[1] = hp[:, DH:D]


_mm_call = pl.pallas_call(
    _mm_body,
    out_shape=[
        jax.ShapeDtypeStruct((N, 1), jnp.float32),
        jax.ShapeDtypeStruct((2, N, DH), jnp.float32),
    ],
    compiler_params=pltpu.CompilerParams(vmem_limit_bytes=100 * 1024 * 1024),
)


K = 2
NBUF = 2 * K
NIT = EPT_AGG // (K * CHUNK)


def _agg_body(hcat_hbm, row_hbm, col_hbm, ew_hbm, agg_hbm,
              idx_v, cidx_v, ew_v, gbuf, shared_agg,
              isem, csem, wsem, gsem, ssem):
    c = lax.axis_index("c")
    s = lax.axis_index("s")
    base = s * EPT_AGG
    htab = hcat_hbm.at[c]
    atab = agg_hbm.at[c]

    pltpu.sync_copy(htab.at[pl.ds(s * NPT, NPT)],
                    shared_agg.at[pl.ds(s * NPT, NPT)])

    @pl.when(s == 0)
    def _init_rem():
        pltpu.sync_copy(htab.at[pl.ds(NT * NPT, NREM)],
                        shared_agg.at[pl.ds(NT * NPT, NREM)])

    plsc.subcore_barrier()

    def _fire(g, ps):
        poff = base + g * (K * CHUNK)
        for j in range(K):
            b = ps * K + j
            off = poff + j * CHUNK

            @pl.when(g >= 2)
            def _drain_prev(b=b):
                pltpu.make_async_copy(
                    gbuf.at[b], shared_agg.at[cidx_v.at[b]], ssem.at[b]
                ).wait()

            pltpu.async_copy(row_hbm.at[pl.ds(off, CHUNK)], idx_v.at[b],
                             isem.at[b])
            pltpu.async_copy(col_hbm.at[pl.ds(off, CHUNK)], cidx_v.at[b],
                             csem.at[b])
            pltpu.async_copy(ew_hbm.at[pl.ds(off, CHUNK)], ew_v.at[b],
                             wsem.at[b])
        for j in range(K):
            b = ps * K + j
            pltpu.make_async_copy(row_hbm.at[pl.ds(base, CHUNK)],
                                  idx_v.at[b], isem.at[b]).wait()
            pltpu.async_copy(htab.at[idx_v.at[b]], gbuf.at[b], gsem.at[b])

    def _process(g, ps):
        poff = base + g * (K * CHUNK)
        for j in range(K):
            b = ps * K + j
            off = poff + j * CHUNK
            pltpu.make_async_copy(htab.at[idx_v.at[b]], gbuf.at[b],
                                  gsem.at[b]).wait()
            pltpu.make_async_copy(ew_hbm.at[pl.ds(base, CHUNK)],
                                  ew_v.at[b], wsem.at[b]).wait()

            def ebody(q, ecarry, b=b):
                wv = ew_v[b, pl.ds(q * 16, 16)]
                jb = q * 16
                for l in range(16):
                    w = wv[l]
                    for f in range(DH // 16):
                        sl = pl.ds(f * 16, 16)
                        gbuf[b, jb + l, sl] = gbuf[b, jb + l, sl] * w
                return ecarry

            lax.fori_loop(0, CHUNK // 16, ebody, 0)
            pltpu.make_async_copy(col_hbm.at[pl.ds(base, CHUNK)],
                                  cidx_v.at[b], csem.at[b]).wait()
            pltpu.async_copy(gbuf.at[b], shared_agg.at[cidx_v.at[b]],
                             ssem.at[b], add=True)

    def it_body(g, carry):
        p = lax.rem(g, 2)

        @pl.when(jnp.logical_and(p == 0, g < NIT))
        def _a0():
            _fire(g, 0)

        @pl.when(jnp.logical_and(p == 1, g < NIT))
        def _a1():
            _fire(g, 1)

        @pl.when(jnp.logical_and(p == 1, g > 0))
        def _b0():
            _process(g - 1, 0)

        @pl.when(jnp.logical_and(p == 0, g > 0))
        def _b1():
            _process(g - 1, 1)

        return carry

    lax.fori_loop(0, NIT + 1, it_body, 0)

    for b in range(NBUF):
        pltpu.make_async_copy(
            gbuf.at[b], shared_agg.at[cidx_v.at[b]], ssem.at[b]).wait()
    plsc.subcore_barrier()

    pltpu.sync_copy(shared_agg.at[pl.ds(s * NPT, NPT)],
                    atab.at[pl.ds(s * NPT, NPT)])

    @pl.when(s == 0)
    def _out_rem():
        pltpu.sync_copy(shared_agg.at[pl.ds(NT * NPT, NREM)],
                        atab.at[pl.ds(NT * NPT, NREM)])


_agg_call = functools.partial(
    pl.kernel,
    out_type=jax.ShapeDtypeStruct((NC, N, DH), jnp.float32),
    mesh=_mesh,
    compiler_params=pltpu.CompilerParams(needs_layout_passes=False),
    scratch_types=[
        pltpu.VMEM((NBUF, CHUNK), jnp.int32),
        pltpu.VMEM((NBUF, CHUNK), jnp.int32),
        pltpu.VMEM((NBUF, CHUNK), jnp.float32),
        pltpu.VMEM((NBUF, CHUNK, DH), jnp.float32),
        pltpu.VMEM_SHARED((N, DH), jnp.float32),
        pltpu.SemaphoreType.DMA((NBUF,)),
        pltpu.SemaphoreType.DMA((NBUF,)),
        pltpu.SemaphoreType.DMA((NBUF,)),
        pltpu.SemaphoreType.DMA((NBUF,)),
        pltpu.SemaphoreType.DMA((NBUF,)),
    ],
)(_agg_body)


def _bn_body(agg_ref, dis_ref, x_ref, b_ref, gamma_ref, beta_ref, out_ref):
    agg = jnp.concatenate([agg_ref[0:N, :], agg_ref[N:2 * N, :]], axis=1)
    y = agg * dis_ref[...] + b_ref[...]
    mean = jnp.mean(y, axis=0, keepdims=True)
    yc = y - mean
    var = jnp.mean(yc * yc, axis=0, keepdims=True)
    out_ref[...] = yc * lax.rsqrt(var + 1e-5) * gamma_ref[...] \
        + beta_ref[...] + x_ref[...]


_bn_call = pl.pallas_call(
    _bn_body,
    out_shape=jax.ShapeDtypeStruct((N, D), jnp.float32),
    compiler_params=pltpu.CompilerParams(vmem_limit_bytes=100 * 1024 * 1024),
)


def kernel(x, edge_index, edge_weight, W, b, gamma, beta):
    row = edge_index[0].astype(jnp.int32)
    col = edge_index[1].astype(jnp.int32)
    ew = edge_weight.astype(jnp.float32)
    pad = E_PAD - E
    row_p = jnp.concatenate([row, jnp.zeros((pad,), jnp.int32)])
    col_p = jnp.concatenate([col, jnp.zeros((pad,), jnp.int32)])
    ew_p = jnp.concatenate([ew, jnp.zeros((pad,), jnp.float32)])

    degp = _deg_call(col_p, ew_p)
    degpt = degp.T
    dis, hcat = _mm_call(degpt, x, W)
    agg = _agg_call(hcat, row_p, col_p, ew_p)
    agg = agg.reshape(NC * N, DH)
    out = _bn_call(agg, dis, x,
                   b.reshape(1, D), gamma.reshape(1, D), beta.reshape(1, D))
    return out
